# baseline (speedup 1.0000x reference)
import jax
import jax.numpy as jnp
from jax.experimental import pallas as pl
from jax.experimental.pallas import tpu as pltpu

_B = 2000


def _make_body(nsteps, b):
    def body(x_ref, s_ref, o_hbm, ns_hbm, eb, nsb, acc, scale, dsem):
        j = pl.program_id(0)
        k = pl.program_id(1)

        def out_copy(hbm, chunk, slot):
            return pltpu.make_async_copy(
                nsb.at[slot], hbm.at[pl.ds(chunk * b, b)], dsem.at[slot])

        @pl.when(j == 0)
        def _phase0():
            slot = jax.lax.rem(k, 2)
            ns = x_ref[...] + s_ref[...]
            e = jnp.exp(ns)
            colsum = jnp.sum(e, axis=0, keepdims=True)
            acc[...] = jnp.where(k == 0, colsum, acc[...] + colsum)
            eb[pl.ds(k * b, b), :] = e

            @pl.when(k >= 2)
            def _drain():
                out_copy(ns_hbm, k - 2, slot).wait()

            nsb[slot] = ns
            out_copy(ns_hbm, k, slot).start()

        @pl.when(j == 1)
        def _phase1():
            slot = jax.lax.rem(k, 2)

            @pl.when(k == 0)
            def _transition():
                out_copy(ns_hbm, nsteps - 2, 0).wait()
                out_copy(ns_hbm, nsteps - 1, 1).wait()
                scale[...] = 2.0 / acc[...]

            @pl.when(k >= 2)
            def _drain():
                out_copy(o_hbm, k - 2, slot).wait()

            nsb[slot] = eb[pl.ds(k * b, b), :] * scale[...]
            out_copy(o_hbm, k, slot).start()

            @pl.when(k == nsteps - 1)
            def _epilogue():
                out_copy(o_hbm, k - 1, jax.lax.rem(k + 1, 2)).wait()
                out_copy(o_hbm, k, slot).wait()

    return body


def kernel(x, state):
    xt = x.T
    st = state.T
    n, m = xt.shape
    b = _B
    nsteps = n // b
    in_spec = pl.BlockSpec((b, m), lambda j, k: (k * (1 - j), 0))
    any_spec = pl.BlockSpec(memory_space=pl.ANY)
    out, ns = pl.pallas_call(
        _make_body(nsteps, b),
        grid=(2, nsteps),
        in_specs=[in_spec, in_spec],
        out_specs=[any_spec, any_spec],
        out_shape=[
            jax.ShapeDtypeStruct((n, m), xt.dtype),
            jax.ShapeDtypeStruct((n, m), xt.dtype),
        ],
        scratch_shapes=[
            pltpu.VMEM((n, m), jnp.float32),
            pltpu.VMEM((2, b, m), jnp.float32),
            pltpu.VMEM((1, m), jnp.float32),
            pltpu.VMEM((1, m), jnp.float32),
            pltpu.SemaphoreType.DMA((2,)),
        ],
    )(xt, st)
    return (out.T, ns.T)


# speedup vs baseline: 1.2918x; 1.2918x over previous
import jax
import jax.numpy as jnp
from jax.experimental import pallas as pl
from jax.experimental.pallas import tpu as pltpu

_B = 4000


def _make_body(nsteps, b):
    def body(x_ref, s_ref, o_hbm, ns_hbm, eb, ring, acc, scale, dsem):
        j = pl.program_id(0)
        k = pl.program_id(1)

        def out_copy(hbm, chunk, slot):
            return pltpu.make_async_copy(
                ring.at[slot], hbm.at[pl.ds(chunk * b, b)], dsem.at[slot])

        @pl.when(j == 0)
        def _phase0():
            slot = jax.lax.rem(k, 2)
            ns = x_ref[...] + s_ref[...]
            e = jnp.exp(ns)
            colsum = jnp.sum(e, axis=0, keepdims=True)
            acc[...] = jnp.where(k == 0, colsum, acc[...] + colsum)
            eb[pl.ds(k * b, b), :] = e.astype(jnp.bfloat16)

            @pl.when(k >= 2)
            def _drain():
                out_copy(ns_hbm, k - 2, slot).wait()

            ring[slot] = ns
            out_copy(ns_hbm, k, slot).start()

        @pl.when(j == 1)
        def _phase1():
            slot = jax.lax.rem(k, 2)

            @pl.when(k == 0)
            def _transition():
                out_copy(ns_hbm, nsteps - 2, 0).wait()
                out_copy(ns_hbm, nsteps - 1, 1).wait()
                scale[...] = 2.0 / acc[...]

            @pl.when(k >= 2)
            def _drain():
                out_copy(o_hbm, k - 2, slot).wait()

            ring[slot] = eb[pl.ds(k * b, b), :].astype(jnp.float32) * scale[...]
            out_copy(o_hbm, k, slot).start()

            @pl.when(k == nsteps - 1)
            def _epilogue():
                out_copy(o_hbm, k - 1, jax.lax.rem(k + 1, 2)).wait()
                out_copy(o_hbm, k, slot).wait()

    return body


def kernel(x, state):
    xt = x.T
    st = state.T
    n, m = xt.shape
    b = _B
    nsteps = n // b
    in_spec = pl.BlockSpec((b, m), lambda j, k: (k * (1 - j), 0))
    any_spec = pl.BlockSpec(memory_space=pl.ANY)
    out, ns = pl.pallas_call(
        _make_body(nsteps, b),
        grid=(2, nsteps),
        in_specs=[in_spec, in_spec],
        out_specs=[any_spec, any_spec],
        out_shape=[
            jax.ShapeDtypeStruct((n, m), xt.dtype),
            jax.ShapeDtypeStruct((n, m), xt.dtype),
        ],
        scratch_shapes=[
            pltpu.VMEM((n, m), jnp.bfloat16),
            pltpu.VMEM((2, b, m), jnp.float32),
            pltpu.VMEM((1, m), jnp.float32),
            pltpu.VMEM((1, m), jnp.float32),
            pltpu.SemaphoreType.DMA((2,)),
        ],
    )(xt, st)
    return (out.T, ns.T)


# speedup vs baseline: 1.3415x; 1.0384x over previous
import jax
import jax.numpy as jnp
from jax.experimental import pallas as pl
from jax.experimental.pallas import tpu as pltpu

_B0 = 4000
_B1 = 10000


def _make_body(n, b0, b1):
    ns0 = n // b0
    ns1 = n // b1

    def body(x_ref, s_ref, o_hbm, ns_hbm, eb, ring, acc, scale, dsem):
        i = pl.program_id(0)

        def ns_copy(chunk, slot):
            return pltpu.make_async_copy(
                ring.at[slot, pl.ds(0, b0)], ns_hbm.at[pl.ds(chunk * b0, b0)],
                dsem.at[slot])

        def o_copy(chunk, slot):
            return pltpu.make_async_copy(
                ring.at[slot], o_hbm.at[pl.ds(chunk * b1, b1)], dsem.at[slot])

        @pl.when(i < ns0)
        def _phase0():
            slot = jax.lax.rem(i, 2)
            ns = x_ref[...] + s_ref[...]
            e = jnp.exp(ns)
            colsum = jnp.sum(e, axis=0, keepdims=True)
            acc[...] = jnp.where(i == 0, colsum, acc[...] + colsum)
            eb[pl.ds(i * b0, b0), :] = e.astype(jnp.bfloat16)

            @pl.when(i >= 2)
            def _drain():
                ns_copy(i - 2, slot).wait()

            ring[slot, pl.ds(0, b0)] = ns
            ns_copy(i, slot).start()

        @pl.when(i >= ns0)
        def _phase1():
            k = i - ns0
            slot = jax.lax.rem(k, 2)

            @pl.when(k == 0)
            def _transition():
                ns_copy(ns0 - 2, jax.lax.rem(ns0 - 2, 2)).wait()
                ns_copy(ns0 - 1, jax.lax.rem(ns0 - 1, 2)).wait()
                scale[...] = 2.0 / acc[...]

            @pl.when(k >= 2)
            def _drain():
                o_copy(k - 2, slot).wait()

            ring[slot] = eb[pl.ds(k * b1, b1), :].astype(jnp.float32) * scale[...]
            o_copy(k, slot).start()

            @pl.when(k == ns1 - 1)
            def _epilogue():
                o_copy(k - 1, jax.lax.rem(k + 1, 2)).wait()
                o_copy(k, slot).wait()

    return body


def kernel(x, state):
    xt = x.T
    st = state.T
    n, m = xt.shape
    b0, b1 = _B0, _B1
    ns0 = n // b0
    ns1 = n // b1
    in_spec = pl.BlockSpec((b0, m), lambda i: (jnp.minimum(i, ns0 - 1), 0))
    any_spec = pl.BlockSpec(memory_space=pl.ANY)
    out, ns = pl.pallas_call(
        _make_body(n, b0, b1),
        grid=(ns0 + ns1,),
        in_specs=[in_spec, in_spec],
        out_specs=[any_spec, any_spec],
        out_shape=[
            jax.ShapeDtypeStruct((n, m), xt.dtype),
            jax.ShapeDtypeStruct((n, m), xt.dtype),
        ],
        scratch_shapes=[
            pltpu.VMEM((n, m), jnp.bfloat16),
            pltpu.VMEM((2, b1, m), jnp.float32),
            pltpu.VMEM((1, m), jnp.float32),
            pltpu.VMEM((1, m), jnp.float32),
            pltpu.SemaphoreType.DMA((2,)),
        ],
    )(xt, st)
    return (out.T, ns.T)


# speedup vs baseline: 1.3577x; 1.0121x over previous
import jax
import jax.numpy as jnp
from jax.experimental import pallas as pl
from jax.experimental.pallas import tpu as pltpu

_B0 = 4000
_B1 = 10000


def _make_body(n, b0, b1):
    ns0 = n // b0
    ns1 = n // b1

    def body(x_ref, s_ref, o_hbm, ns_hbm, eb, ring, acc, scale, dsem):
        i = pl.program_id(0)

        def ns_copy(chunk, slot):
            return pltpu.make_async_copy(
                ring.at[slot, pl.ds(0, b0)], ns_hbm.at[pl.ds(chunk * b0, b0)],
                dsem.at[slot])

        def o_copy(chunk, slot):
            return pltpu.make_async_copy(
                ring.at[slot], o_hbm.at[pl.ds(chunk * b1, b1)], dsem.at[slot])

        @pl.when(i < ns0)
        def _phase0():
            slot = jax.lax.rem(i, 2)
            ns = x_ref[...] + s_ref[...]
            e = jnp.exp(ns)
            colsum = jnp.sum(e, axis=0, keepdims=True)
            acc[...] = jnp.where(i == 0, colsum, acc[...] + colsum)
            eb[pl.ds(i * b0, b0), :] = e.astype(jnp.bfloat16)

            @pl.when(i >= 2)
            def _drain():
                ns_copy(i - 2, slot).wait()

            ring[slot, pl.ds(0, b0)] = ns
            ns_copy(i, slot).start()

        @pl.when(i >= ns0)
        def _phase1():
            k = i - ns0
            slot = jax.lax.rem(k + ns0, 2)

            @pl.when(k == 0)
            def _transition():
                ns_copy(ns0 - 2, jax.lax.rem(ns0 - 2, 2)).wait()
                scale[...] = 2.0 / acc[...]

            @pl.when(k == 1)
            def _transition2():
                ns_copy(ns0 - 1, jax.lax.rem(ns0 - 1, 2)).wait()

            @pl.when(k >= 2)
            def _drain():
                o_copy(k - 2, slot).wait()

            ring[slot] = eb[pl.ds(k * b1, b1), :].astype(jnp.float32) * scale[...]
            o_copy(k, slot).start()

            @pl.when(k == ns1 - 1)
            def _epilogue():
                o_copy(k - 1, jax.lax.rem(k + ns0 + 1, 2)).wait()
                o_copy(k, slot).wait()

    return body


def kernel(x, state):
    xt = x.T
    st = state.T
    n, m = xt.shape
    b0, b1 = _B0, _B1
    ns0 = n // b0
    ns1 = n // b1
    in_spec = pl.BlockSpec((b0, m), lambda i: (jnp.minimum(i, ns0 - 1), 0))
    any_spec = pl.BlockSpec(memory_space=pl.ANY)
    out, ns = pl.pallas_call(
        _make_body(n, b0, b1),
        grid=(ns0 + ns1,),
        in_specs=[in_spec, in_spec],
        out_specs=[any_spec, any_spec],
        out_shape=[
            jax.ShapeDtypeStruct((n, m), xt.dtype),
            jax.ShapeDtypeStruct((n, m), xt.dtype),
        ],
        scratch_shapes=[
            pltpu.VMEM((n, m), jnp.bfloat16),
            pltpu.VMEM((2, b1, m), jnp.float32),
            pltpu.VMEM((1, m), jnp.float32),
            pltpu.VMEM((1, m), jnp.float32),
            pltpu.SemaphoreType.DMA((2,)),
        ],
    )(xt, st)
    return (out.T, ns.T)
